# baseline (speedup 1.0000x reference)
import functools

import jax
import jax.numpy as jnp
from jax import lax
from jax.experimental import pallas as pl
from jax.experimental.pallas import tpu as pltpu
from jax.experimental.pallas import tpu_sc as plsc

B, N, C, E = 2, 2048, 128, 4
K = N // 2
NC, NS, L = 2, 16, 16
ROWS_PER_TILE = (B * K) // (NC * NS)
ACH = 8
ECH = 2


def _score_topk_body(x_ref, w1_ref, b1_ref, w2_ref, b2_ref, w3_ref, idx_ref):
    x = x_ref[0]
    h = jnp.dot(x, w1_ref[...], preferred_element_type=jnp.float32)
    h = jnp.maximum(h + b1_ref[...], 0.0)
    h = jnp.dot(h, w2_ref[...], preferred_element_type=jnp.float32)
    h = jnp.maximum(h + b2_ref[...], 0.0)
    s_col = jnp.dot(h, w3_ref[...], preferred_element_type=jnp.float32)
    u = jax.lax.bitcast_convert_type(s_col, jnp.int32)
    k_col = jnp.where(u >= 0, u, u ^ jnp.int32(0x7FFFFFFF))
    k_row = jax.lax.transpose(k_col, (1, 0))
    i_lane = jax.lax.broadcasted_iota(jnp.int32, (1, N), 1)

    rank = jnp.zeros((1, N), jnp.int32)
    jc = 256
    for c in range(N // jc):
        k_j = jax.lax.slice(k_col, (c * jc, 0), ((c + 1) * jc, 1))
        j_sub = jax.lax.broadcasted_iota(jnp.int32, (jc, 1), 0) + c * jc
        beats = (k_j > k_row) | ((k_j == k_row) & (j_sub < i_lane))
        rank = rank + jnp.sum(beats.astype(jnp.int32), axis=0, keepdims=True)

    parts = []
    rc = 256
    for c in range(K // rc):
        r_sub = jax.lax.broadcasted_iota(jnp.int32, (rc, 1), 0) + c * rc
        eq = rank == r_sub
        parts.append(jnp.sum(jnp.where(eq, i_lane, 0), axis=1, keepdims=True))
    idx_ref[...] = jnp.concatenate(parts, axis=0)[None]


def _score_topk(x, W1, b1, W2, b2, W3):
    return pl.pallas_call(
        _score_topk_body,
        grid=(B,),
        in_specs=[
            pl.BlockSpec((1, N, C), lambda b: (b, 0, 0)),
            pl.BlockSpec((C, 64), lambda b: (0, 0)),
            pl.BlockSpec((1, 64), lambda b: (0, 0)),
            pl.BlockSpec((64, 16), lambda b: (0, 0)),
            pl.BlockSpec((1, 16), lambda b: (0, 0)),
            pl.BlockSpec((16, 1), lambda b: (0, 0)),
        ],
        out_specs=pl.BlockSpec((1, K, 1), lambda b: (b, 0, 0)),
        out_shape=jax.ShapeDtypeStruct((B, K, 1), jnp.int32),
    )(x, W1, b1.reshape(1, 64), W2, b2.reshape(1, 16), W3)


def _gather_body(x_hbm, adj_hbm, edge_hbm, idx_hbm, outf_hbm, outa_hbm, oute_hbm,
                 idx_v, myidx_v, midx4_v, feat_v, arow_v, aout_v,
                 erow_v, eout_v, sem_f, sem_a, sem_a1, sem_ao, sem_ao1,
                 sem_e, sem_e1, sem_o, sem_o1):
    cid = lax.axis_index("c")
    sid = lax.axis_index("s")
    wid = cid * NS + sid
    b = wid // NS
    lr0 = (wid % NS) * ROWS_PER_TILE

    iota = lax.iota(jnp.int32, L)
    pltpu.sync_copy(idx_hbm.at[b], idx_v)
    pltpu.sync_copy(idx_hbm.at[b, pl.ds(lr0, ROWS_PER_TILE)], myidx_v)

    fcopy = pltpu.async_copy(x_hbm.at[b].at[myidx_v], feat_v, sem_f)

    for t in range(ROWS_PER_TILE // L):
        v = myidx_v[pl.ds(t * L, L)]
        gi = t * L + iota
        pos = (gi // ECH) * 8 + (gi % ECH)
        plsc.store_scatter(midx4_v, [pos], v)

    nach = ROWS_PER_TILE // ACH
    sems_a = [sem_a, sem_a1]
    sems_ao = [sem_ao, sem_ao1]
    acp = [pltpu.async_copy(adj_hbm.at[b].at[myidx_v.at[pl.ds(0, ACH)]],
                            arow_v.at[0], sem_a), None]
    aocp = [None, None]
    for ch in range(nach):
        p = ch & 1
        if ch + 1 < nach:
            acp[1 - p] = pltpu.async_copy(
                adj_hbm.at[b].at[myidx_v.at[pl.ds((ch + 1) * ACH, ACH)]],
                arow_v.at[1 - p], sems_a[1 - p])
        acp[p].wait()
        if aocp[p] is not None:
            aocp[p].wait()

        @plsc.parallel_loop(0, ACH * (K // L), 1, unroll=8)
        def _(t):
            g = t >> 3
            r = t & 7
            col = idx_v[pl.ds(g * L, L)]
            aout_v[p, r, pl.ds(g * L, L)] = plsc.load_gather(
                arow_v,
                [jnp.full((L,), p, jnp.int32),
                 jnp.full((L,), r, jnp.int32), col])

        aocp[p] = pltpu.async_copy(
            aout_v.at[p], outa_hbm.at[b, pl.ds(lr0 + ch * ACH, ACH)],
            sems_ao[p])
    for p in range(2):
        if aocp[p] is not None:
            aocp[p].wait()

    fcopy.wait()
    pltpu.sync_copy(feat_v, outf_hbm.at[b, pl.ds(lr0, ROWS_PER_TILE)])

    nch = ROWS_PER_TILE // ECH
    sems_e = [sem_e, sem_e1]
    sems_o = [sem_o, sem_o1]
    ecp = [pltpu.async_copy(edge_hbm.at[b].at[midx4_v.at[pl.ds(0, ECH)]],
                            erow_v.at[0], sem_e), None]
    ocp = [None, None]
    for ch in range(nch):
        p = ch & 1
        if ch + 1 < nch:
            ecp[1 - p] = pltpu.async_copy(
                edge_hbm.at[b].at[midx4_v.at[pl.ds((ch + 1) * 8, ECH)]],
                erow_v.at[1 - p], sems_e[1 - p])
        ecp[p].wait()
        if ocp[p] is not None:
            ocp[p].wait()

        @plsc.parallel_loop(0, ECH * E * (K // L), 1, unroll=16)
        def _(t):
            g = t >> 3
            r = (t >> 2) & 1
            e = t & 3
            col = idx_v[pl.ds(g * L, L)]
            eout_v[p, r, e, pl.ds(g * L, L)] = plsc.load_gather(
                erow_v,
                [jnp.full((L,), p, jnp.int32),
                 jnp.full((L,), r, jnp.int32),
                 jnp.full((L,), e, jnp.int32), col])

        ocp[p] = pltpu.async_copy(
            eout_v.at[p], oute_hbm.at[b, pl.ds(lr0 + ch * ECH, ECH)],
            sems_o[p])
    for p in range(2):
        if ocp[p] is not None:
            ocp[p].wait()


def _gather_call():
  return functools.partial(
    pl.kernel,
    mesh=plsc.VectorSubcoreMesh(core_axis_name="c", subcore_axis_name="s"),
    out_type=(
        jax.ShapeDtypeStruct((B, K, C), jnp.float32),
        jax.ShapeDtypeStruct((B, K, K), jnp.float32),
        jax.ShapeDtypeStruct((B, K, E, K), jnp.float32),
    ),
    scratch_types=[
        pltpu.VMEM((K,), jnp.int32),
        pltpu.VMEM((ROWS_PER_TILE,), jnp.int32),
        pltpu.VMEM(((ROWS_PER_TILE // ECH) * 8,), jnp.int32),
        pltpu.VMEM((ROWS_PER_TILE, C), jnp.float32),
        pltpu.VMEM((2, ACH, N), jnp.float32),
        pltpu.VMEM((2, ACH, K), jnp.float32),
        pltpu.VMEM((2, ECH, E, N), jnp.float32),
        pltpu.VMEM((2, ECH, E, K), jnp.float32),
    ] + [pltpu.SemaphoreType.DMA] * 9,
    compiler_params=pltpu.CompilerParams(needs_layout_passes=False),
  )


def kernel(x, adjacency, edge_features, superpoint_centroids,
           W1, b1, W2, b2, W3, b3):
    del superpoint_centroids, b3
    idx = _score_topk(x, W1, b1, W2, b2, W3).reshape(B, K)
    gather = _gather_call()(_gather_body)
    pooled_f, pooled_a, pooled_e = gather(
        x, adjacency, jnp.swapaxes(edge_features, 2, 3), idx)
    return (pooled_f, pooled_a, jnp.swapaxes(pooled_e, 2, 3))


# speedup vs baseline: 1.0201x; 1.0201x over previous
import functools

import jax
import jax.numpy as jnp
from jax import lax
from jax.experimental import pallas as pl
from jax.experimental.pallas import tpu as pltpu
from jax.experimental.pallas import tpu_sc as plsc

B, N, C, E = 2, 2048, 128, 4
K = N // 2
NC, NS, L = 2, 16, 16
ROWS_PER_TILE = (B * K) // (NC * NS)
ACH = 8
ECH = 2


def _score_topk_body(x_ref, w1_ref, b1_ref, w2_ref, b2_ref, w3_ref, idx_ref):
    for bb in range(B):
        x = x_ref[bb]
        h = jnp.dot(x, w1_ref[...], preferred_element_type=jnp.float32)
        h = jnp.maximum(h + b1_ref[...], 0.0)
        h = jnp.dot(h, w2_ref[...], preferred_element_type=jnp.float32)
        h = jnp.maximum(h + b2_ref[...], 0.0)
        s_col = jnp.dot(h, w3_ref[...], preferred_element_type=jnp.float32)
        u = jax.lax.bitcast_convert_type(s_col, jnp.int32)
        k_col = jnp.where(u >= 0, u, u ^ jnp.int32(0x7FFFFFFF))
        k_row = jax.lax.transpose(k_col, (1, 0))
        i_lane = jax.lax.broadcasted_iota(jnp.int32, (1, N), 1)

        rank = jnp.zeros((1, N), jnp.int32)
        jc = 256
        for c in range(N // jc):
            k_j = jax.lax.slice(k_col, (c * jc, 0), ((c + 1) * jc, 1))
            j_sub = jax.lax.broadcasted_iota(jnp.int32, (jc, 1), 0) + c * jc
            beats = (k_j > k_row) | ((k_j == k_row) & (j_sub < i_lane))
            rank = rank + jnp.sum(beats.astype(jnp.int32), axis=0,
                                  keepdims=True)

        parts = []
        rc = 256
        for c in range(K // rc):
            r_sub = jax.lax.broadcasted_iota(jnp.int32, (rc, 1), 0) + c * rc
            eq = rank == r_sub
            parts.append(jnp.sum(jnp.where(eq, i_lane, 0), axis=1,
                                 keepdims=True))
        idx_ref[bb] = jnp.concatenate(parts, axis=0)


def _score_topk(x, W1, b1, W2, b2, W3):
    return pl.pallas_call(
        _score_topk_body,
        grid=(1,),
        in_specs=[
            pl.BlockSpec((B, N, C), lambda b: (0, 0, 0)),
            pl.BlockSpec((C, 64), lambda b: (0, 0)),
            pl.BlockSpec((1, 64), lambda b: (0, 0)),
            pl.BlockSpec((64, 16), lambda b: (0, 0)),
            pl.BlockSpec((1, 16), lambda b: (0, 0)),
            pl.BlockSpec((16, 1), lambda b: (0, 0)),
        ],
        out_specs=pl.BlockSpec((B, K, 1), lambda b: (0, 0, 0)),
        out_shape=jax.ShapeDtypeStruct((B, K, 1), jnp.int32),
    )(x, W1, b1.reshape(1, 64), W2, b2.reshape(1, 16), W3)


def _gather_body(x_hbm, adj_hbm, edge_hbm, idx_hbm, outf_hbm, outa_hbm, oute_hbm,
                 idx_v, myidx_v, midx4_v, feat_v, arow_v, aout_v,
                 erow_v, eout_v, sem_f, sem_a, sem_a1, sem_ao, sem_ao1,
                 sem_e, sem_e1, sem_o, sem_o1):
    cid = lax.axis_index("c")
    sid = lax.axis_index("s")
    wid = cid * NS + sid
    b = wid // NS
    lr0 = (wid % NS) * ROWS_PER_TILE

    iota = lax.iota(jnp.int32, L)
    pltpu.sync_copy(idx_hbm.at[b], idx_v)
    pltpu.sync_copy(idx_hbm.at[b, pl.ds(lr0, ROWS_PER_TILE)], myidx_v)

    fcopy = pltpu.async_copy(x_hbm.at[b].at[myidx_v], feat_v, sem_f)

    for t in range(ROWS_PER_TILE // L):
        v = myidx_v[pl.ds(t * L, L)]
        gi = t * L + iota
        pos = (gi // ECH) * 8 + (gi % ECH)
        plsc.store_scatter(midx4_v, [pos], v)

    sems_e = [sem_e, sem_e1]
    sems_o = [sem_o, sem_o1]
    ecp = [pltpu.async_copy(edge_hbm.at[b].at[midx4_v.at[pl.ds(0, ECH)]],
                            erow_v.at[0], sem_e), None]

    nach = ROWS_PER_TILE // ACH
    sems_a = [sem_a, sem_a1]
    sems_ao = [sem_ao, sem_ao1]
    acp = [pltpu.async_copy(adj_hbm.at[b].at[myidx_v.at[pl.ds(0, ACH)]],
                            arow_v.at[0], sem_a), None]
    aocp = [None, None]
    for ch in range(nach):
        p = ch & 1
        if ch + 1 < nach:
            acp[1 - p] = pltpu.async_copy(
                adj_hbm.at[b].at[myidx_v.at[pl.ds((ch + 1) * ACH, ACH)]],
                arow_v.at[1 - p], sems_a[1 - p])
        acp[p].wait()
        if aocp[p] is not None:
            aocp[p].wait()

        @plsc.parallel_loop(0, ACH * (K // L), 1, unroll=8)
        def _(t):
            g = t >> 3
            r = t & 7
            col = idx_v[pl.ds(g * L, L)]
            aout_v[p, r, pl.ds(g * L, L)] = plsc.load_gather(
                arow_v,
                [jnp.full((L,), p, jnp.int32),
                 jnp.full((L,), r, jnp.int32), col])

        aocp[p] = pltpu.async_copy(
            aout_v.at[p], outa_hbm.at[b, pl.ds(lr0 + ch * ACH, ACH)],
            sems_ao[p])
    for p in range(2):
        if aocp[p] is not None:
            aocp[p].wait()

    fcopy.wait()
    pltpu.sync_copy(feat_v, outf_hbm.at[b, pl.ds(lr0, ROWS_PER_TILE)])

    nch = ROWS_PER_TILE // ECH
    ocp = [None, None]
    for ch in range(nch):
        p = ch & 1
        if ch + 1 < nch:
            ecp[1 - p] = pltpu.async_copy(
                edge_hbm.at[b].at[midx4_v.at[pl.ds((ch + 1) * 8, ECH)]],
                erow_v.at[1 - p], sems_e[1 - p])
        ecp[p].wait()
        if ocp[p] is not None:
            ocp[p].wait()

        @plsc.parallel_loop(0, ECH * E * (K // L), 1, unroll=8)
        def _(t):
            g = t >> 3
            r = (t >> 2) & 1
            e = t & 3
            col = idx_v[pl.ds(g * L, L)]
            eout_v[p, r, e, pl.ds(g * L, L)] = plsc.load_gather(
                erow_v,
                [jnp.full((L,), p, jnp.int32),
                 jnp.full((L,), r, jnp.int32),
                 jnp.full((L,), e, jnp.int32), col])

        ocp[p] = pltpu.async_copy(
            eout_v.at[p], oute_hbm.at[b, pl.ds(lr0 + ch * ECH, ECH)],
            sems_o[p])
    for p in range(2):
        if ocp[p] is not None:
            ocp[p].wait()


def _gather_call():
  return functools.partial(
    pl.kernel,
    mesh=plsc.VectorSubcoreMesh(core_axis_name="c", subcore_axis_name="s"),
    out_type=(
        jax.ShapeDtypeStruct((B, K, C), jnp.float32),
        jax.ShapeDtypeStruct((B, K, K), jnp.float32),
        jax.ShapeDtypeStruct((B, K, E, K), jnp.float32),
    ),
    scratch_types=[
        pltpu.VMEM((K,), jnp.int32),
        pltpu.VMEM((ROWS_PER_TILE,), jnp.int32),
        pltpu.VMEM(((ROWS_PER_TILE // ECH) * 8,), jnp.int32),
        pltpu.VMEM((ROWS_PER_TILE, C), jnp.float32),
        pltpu.VMEM((2, ACH, N), jnp.float32),
        pltpu.VMEM((2, ACH, K), jnp.float32),
        pltpu.VMEM((2, ECH, E, N), jnp.float32),
        pltpu.VMEM((2, ECH, E, K), jnp.float32),
    ] + [pltpu.SemaphoreType.DMA] * 9,
    compiler_params=pltpu.CompilerParams(needs_layout_passes=False),
  )


def kernel(x, adjacency, edge_features, superpoint_centroids,
           W1, b1, W2, b2, W3, b3):
    del superpoint_centroids, b3
    idx = _score_topk(x, W1, b1, W2, b2, W3).reshape(B, K)
    gather = _gather_call()(_gather_body)
    pooled_f, pooled_a, pooled_e = gather(
        x, adjacency, jnp.swapaxes(edge_features, 2, 3), idx)
    return (pooled_f, pooled_a, jnp.swapaxes(pooled_e, 2, 3))
